# baseline (device time: 124463 ns/iter reference)
import jax
import jax.numpy as jnp
from jax import lax
from jax.experimental import pallas as pl
from jax.experimental.pallas import tpu as pltpu

N_DEV = 4
B_PER = 2
SQ = 512
HQ = 32
R = HQ // N_DEV
DH = 64
DM = 768
WINDOW = 128
NBUF = 4

_SLOT_OFF = (0, 3, 2, 1)
_UNITS = [(b, slot) for slot in (0, 3, 1, 2) for b in range(B_PER)]


def _fused(x, wq, wo, k_hbm, v_hbm, mask):
    def body(x_ref, wq_ref, wo_ref, k_ref, v_ref, m_ref, out_ref,
             commq, commo, ctx_sc, kbuf, vbuf,
             send_q, recv_q, send_o, recv_o, ksem, vsem):
        my = lax.axis_index("i")
        left = lax.rem(my - 1 + N_DEV, N_DEV)
        right = lax.rem(my + 1, N_DEV)

        def kv_copy(unit, tensor_ref, buf, sems):
            b, slot = _UNITS[unit]
            origin = lax.rem(my + _SLOT_OFF[slot], N_DEV)
            return pltpu.make_async_copy(
                tensor_ref.at[my * B_PER + b, :, pl.ds(origin * R * DH, R * DH)],
                buf.at[unit % NBUF],
                sems.at[unit % NBUF],
            )

        for u in range(NBUF):
            kv_copy(u, k_ref, kbuf, ksem).start()
            kv_copy(u, v_ref, vbuf, vsem).start()

        commq[0] = wq_ref[...]
        commo[0] = wo_ref[...]

        barrier_sem = pltpu.get_barrier_semaphore()
        for nbr in (left, right):
            pl.semaphore_signal(
                barrier_sem, inc=1,
                device_id=(nbr,), device_id_type=pl.DeviceIdType.MESH,
            )
        pl.semaphore_wait(barrier_sem, 2)

        def rdma(ref, s_sems, r_sems, src_slot, dst_slot, sem, dst):
            return pltpu.make_async_remote_copy(
                src_ref=ref.at[src_slot],
                dst_ref=ref.at[dst_slot],
                send_sem=s_sems.at[sem],
                recv_sem=r_sems.at[sem],
                device_id=(dst,),
                device_id_type=pl.DeviceIdType.MESH,
            )

        def compute(unit, init):
            b, slot = _UNITS[unit]
            kv_copy(unit, k_ref, kbuf, ksem).wait()
            kv_copy(unit, v_ref, vbuf, vsem).wait()
            kb = kbuf[unit % NBUF]
            vb = vbuf[unit % NBUF]
            wq_s = commq[slot]
            wo_s = commo[slot]
            qb = lax.dot_general(
                x_ref[b], wq_s, (((1,), (0,)), ((), ())),
                preferred_element_type=jnp.float32,
            )
            qb = (qb * 0.125).astype(jnp.bfloat16)
            for r in range(R):
                cols = pl.ds(r * DH, DH)
                qr = qb[:, r * DH:(r + 1) * DH]
                kr = kb[:, r * DH:(r + 1) * DH].astype(jnp.bfloat16)
                s = lax.dot_general(
                    qr, kr, (((1,), (1,)), ((), ())),
                    preferred_element_type=jnp.float32,
                )
                w = jnp.exp(s.astype(jnp.bfloat16)) * m_ref[...]
                denom = jnp.sum(w.astype(jnp.float32), axis=-1,
                                keepdims=True)
                w = w * (1.0 / denom).astype(jnp.bfloat16)
                vr = vb[:, r * DH:(r + 1) * DH].astype(jnp.bfloat16)
                ctx_sc[:, r * DH:(r + 1) * DH] = lax.dot_general(
                    w, vr, (((1,), (0,)), ((), ())),
                    preferred_element_type=jnp.float32,
                ).astype(jnp.bfloat16)
            contrib = lax.dot_general(
                ctx_sc[...], wo_s, (((1,), (0,)), ((), ())),
                preferred_element_type=jnp.float32,
            )
            if init:
                out_ref[b] = contrib
            else:
                out_ref[b] = out_ref[b] + contrib
            nxt = unit + NBUF
            if nxt < len(_UNITS):
                kv_copy(nxt, k_ref, kbuf, ksem).start()
                kv_copy(nxt, v_ref, vbuf, vsem).start()

        r0 = [
            rdma(commq, send_q, recv_q, 0, 1, 0, right),
            rdma(commo, send_o, recv_o, 0, 1, 0, right),
            rdma(commq, send_q, recv_q, 0, 3, 2, left),
            rdma(commo, send_o, recv_o, 0, 3, 2, left),
        ]
        for r in r0:
            r.start()
        compute(0, init=True)
        compute(1, init=True)
        for r in r0:
            r.wait()

        r1 = [
            rdma(commq, send_q, recv_q, 1, 2, 1, right),
            rdma(commo, send_o, recv_o, 1, 2, 1, right),
        ]
        for r in r1:
            r.start()
        compute(2, init=False)
        compute(3, init=False)
        compute(4, init=False)
        compute(5, init=False)
        for r in r1:
            r.wait()
        compute(6, init=False)
        compute(7, init=False)

    return pl.pallas_call(
        body,
        out_shape=jax.ShapeDtypeStruct((B_PER, SQ, DM), jnp.float32),
        in_specs=[
            pl.BlockSpec(memory_space=pltpu.VMEM),
            pl.BlockSpec(memory_space=pltpu.VMEM),
            pl.BlockSpec(memory_space=pltpu.VMEM),
            pl.BlockSpec(memory_space=pl.ANY),
            pl.BlockSpec(memory_space=pl.ANY),
            pl.BlockSpec(memory_space=pltpu.VMEM),
        ],
        out_specs=pl.BlockSpec(memory_space=pltpu.VMEM),
        scratch_shapes=[
            pltpu.VMEM((N_DEV, DM, R * DH), jnp.bfloat16),
            pltpu.VMEM((N_DEV, R * DH, DM), jnp.bfloat16),
            pltpu.VMEM((SQ, R * DH), jnp.bfloat16),
            pltpu.VMEM((NBUF, SQ, R * DH), jnp.float32),
            pltpu.VMEM((NBUF, SQ, R * DH), jnp.float32),
            pltpu.SemaphoreType.DMA((3,)),
            pltpu.SemaphoreType.DMA((3,)),
            pltpu.SemaphoreType.DMA((3,)),
            pltpu.SemaphoreType.DMA((3,)),
            pltpu.SemaphoreType.DMA((NBUF,)),
            pltpu.SemaphoreType.DMA((NBUF,)),
        ],
        compiler_params=pltpu.CompilerParams(collective_id=0),
    )(x, wq, wo, k_hbm, v_hbm, mask)


def kernel(x, Wq, K_ext, V_ext, Wo):
    x_bf = x.astype(jnp.bfloat16)

    k2 = K_ext.reshape(8, SQ, HQ * DH)
    v2 = V_ext.reshape(8, SQ, HQ * DH)

    idx = jnp.arange(SQ)
    mask = (jnp.abs(idx[:, None] - idx[None, :]) <= WINDOW).astype(jnp.bfloat16)

    return _fused(
        x_bf,
        Wq.astype(jnp.bfloat16),
        Wo.astype(jnp.bfloat16),
        k2, v2, mask,
    )


# device time: 71914 ns/iter; 1.7307x vs baseline; 1.7307x over previous
import jax
import jax.numpy as jnp
from jax import lax
from jax.experimental import pallas as pl
from jax.experimental.pallas import tpu as pltpu

N_DEV = 4
B_PER = 2
SQ = 512
HQ = 32
R = HQ // N_DEV
DH = 64
DM = 768
WINDOW = 128

_SLOT_OFF = (0, 3, 2, 1)


def _fused(x, wq, wo, ks, vs, mask):
    def body(x_ref, wq_ref, wo_ref, k_ref, v_ref, m_ref, out_ref,
             commq, commo, ctx_sc, send_q, recv_q, send_o, recv_o):
        my = lax.axis_index("i")
        left = lax.rem(my - 1 + N_DEV, N_DEV)
        right = lax.rem(my + 1, N_DEV)

        commq[0] = wq_ref[...]
        commo[0] = wo_ref[...]

        barrier_sem = pltpu.get_barrier_semaphore()
        for nbr in (left, right):
            pl.semaphore_signal(
                barrier_sem, inc=1,
                device_id=(nbr,), device_id_type=pl.DeviceIdType.MESH,
            )
        pl.semaphore_wait(barrier_sem, 2)

        def rdma(ref, s_sems, r_sems, src_slot, dst_slot, sem, dst):
            return pltpu.make_async_remote_copy(
                src_ref=ref.at[src_slot],
                dst_ref=ref.at[dst_slot],
                send_sem=s_sems.at[sem],
                recv_sem=r_sems.at[sem],
                device_id=(dst,),
                device_id_type=pl.DeviceIdType.MESH,
            )

        def compute(slot, init):
            origin = lax.rem(my + _SLOT_OFF[slot], N_DEV)
            wq_s = commq[slot]
            wo_s = commo[slot]
            for b in range(B_PER):
                qb = lax.dot_general(
                    x_ref[b], wq_s, (((1,), (0,)), ((), ())),
                    preferred_element_type=jnp.float32,
                )
                qb = (qb * 0.125).astype(jnp.bfloat16)
                for r in range(R):
                    qr = qb[:, r * DH:(r + 1) * DH]
                    s = lax.dot_general(
                        qr, k_ref[b, origin, r], (((1,), (1,)), ((), ())),
                        preferred_element_type=jnp.float32,
                    )
                    w = jnp.exp(s.astype(jnp.bfloat16)) * m_ref[...]
                    denom = jnp.sum(w.astype(jnp.float32), axis=-1,
                                    keepdims=True)
                    w = w * (1.0 / denom).astype(jnp.bfloat16)
                    ctx_sc[:, r * DH:(r + 1) * DH] = lax.dot_general(
                        w, v_ref[b, origin, r], (((1,), (0,)), ((), ())),
                        preferred_element_type=jnp.float32,
                    ).astype(jnp.bfloat16)
                contrib = lax.dot_general(
                    ctx_sc[...], wo_s, (((1,), (0,)), ((), ())),
                    preferred_element_type=jnp.float32,
                )
                if init:
                    out_ref[b] = contrib
                else:
                    out_ref[b] = out_ref[b] + contrib

        r0 = [
            rdma(commq, send_q, recv_q, 0, 1, 0, right),
            rdma(commo, send_o, recv_o, 0, 1, 0, right),
            rdma(commq, send_q, recv_q, 0, 3, 2, left),
            rdma(commo, send_o, recv_o, 0, 3, 2, left),
        ]
        for r in r0:
            r.start()
        compute(0, init=True)
        for r in r0:
            r.wait()

        r1 = [
            rdma(commq, send_q, recv_q, 1, 2, 1, right),
            rdma(commo, send_o, recv_o, 1, 2, 1, right),
        ]
        for r in r1:
            r.start()
        compute(3, init=False)
        compute(1, init=False)
        for r in r1:
            r.wait()
        compute(2, init=False)

    return pl.pallas_call(
        body,
        out_shape=jax.ShapeDtypeStruct((B_PER, SQ, DM), jnp.float32),
        in_specs=[pl.BlockSpec(memory_space=pltpu.VMEM)] * 6,
        out_specs=pl.BlockSpec(memory_space=pltpu.VMEM),
        scratch_shapes=[
            pltpu.VMEM((N_DEV, DM, R * DH), jnp.bfloat16),
            pltpu.VMEM((N_DEV, R * DH, DM), jnp.bfloat16),
            pltpu.VMEM((SQ, R * DH), jnp.bfloat16),
            pltpu.SemaphoreType.DMA((3,)),
            pltpu.SemaphoreType.DMA((3,)),
            pltpu.SemaphoreType.DMA((3,)),
            pltpu.SemaphoreType.DMA((3,)),
        ],
        compiler_params=pltpu.CompilerParams(collective_id=0),
    )(x, wq, wo, ks, vs, mask)


def kernel(x, Wq, K_ext, V_ext, Wo):
    my = lax.axis_index("i")

    x_bf = x.astype(jnp.bfloat16)

    def prep(t):
        t = lax.dynamic_slice_in_dim(t, my * B_PER, B_PER, axis=0)
        t = t.astype(jnp.bfloat16).transpose(0, 2, 1, 3)
        return t.reshape(B_PER, N_DEV, R, SQ, DH)

    ks = prep(K_ext)
    vs = prep(V_ext)

    idx = jnp.arange(SQ)
    mask = (jnp.abs(idx[:, None] - idx[None, :]) <= WINDOW).astype(jnp.bfloat16)

    return _fused(
        x_bf,
        Wq.astype(jnp.bfloat16),
        Wo.astype(jnp.bfloat16),
        ks, vs, mask,
    )
